# baseline (speedup 1.0000x reference)
import jax
import jax.numpy as jnp
from jax import lax
from jax.experimental import pallas as pl
from jax.experimental.pallas import tpu as pltpu
from jax.experimental.pallas import tpu_sc as plsc

BATCH = 4096
SEQ = 200
TOKEN_DIM = 64
VOCAB = 100000
PAD_DIM = 128
NUM_IDX = BATCH * SEQ
NUM_WORKERS = 32
PER_WORKER = NUM_IDX // NUM_WORKERS
CHUNK = 160
N_CHUNKS = PER_WORKER // CHUNK
LANES = 16


def _sc_gather(table_p, flat_code):
    mesh = plsc.VectorSubcoreMesh(core_axis_name="c", subcore_axis_name="s")

    @pl.kernel(
        out_type=jax.ShapeDtypeStruct((BATCH, SEQ, TOKEN_DIM), table_p.dtype),
        mesh=mesh,
        scratch_types=[
            pltpu.VMEM((PER_WORKER,), jnp.int32),
            pltpu.VMEM((CHUNK, PAD_DIM), jnp.float32),
            pltpu.VMEM((CHUNK, PAD_DIM), jnp.float32),
            pltpu.VMEM((CHUNK, TOKEN_DIM), jnp.float32),
            pltpu.VMEM((CHUNK, TOKEN_DIM), jnp.float32),
            pltpu.SemaphoreType.DMA,
            pltpu.SemaphoreType.DMA,
            pltpu.SemaphoreType.DMA,
            pltpu.SemaphoreType.DMA,
        ],
    )
    def gather_kernel(table_hbm, idx_hbm, out3_hbm, idx_all,
                      rows0, rows1, cmp0, cmp1,
                      sem_g0, sem_g1, sem_o0, sem_o1):
        out_hbm = out3_hbm.reshape(NUM_IDX, TOKEN_DIM)
        wid = lax.axis_index("s") * 2 + lax.axis_index("c")
        base = wid * PER_WORKER
        rows = (rows0, rows1)
        cmps = (cmp0, cmp1)
        sem_g = (sem_g0, sem_g1)
        sem_o = (sem_o0, sem_o1)

        pltpu.sync_copy(idx_hbm.at[pl.ds(base, PER_WORKER)], idx_all)

        def start_gather(c, b):
            pltpu.async_copy(
                table_hbm.at[idx_all.at[pl.ds(c * CHUNK, CHUNK)]],
                rows[b], sem_g[b])

        def wait_gather(b):
            pltpu.make_async_copy(
                table_hbm.at[idx_all.at[pl.ds(0, CHUNK)]],
                rows[b], sem_g[b]).wait()

        def compact(b):
            @plsc.parallel_loop(0, CHUNK, unroll=4)
            def _(i):
                for k in range(TOKEN_DIM // LANES):
                    cmps[b][i, pl.ds(k * LANES, LANES)] = (
                        rows[b][i, pl.ds(k * LANES, LANES)])

        def start_out(c, b):
            pltpu.async_copy(cmps[b], out_hbm.at[pl.ds(base + c * CHUNK,
                                                       CHUNK)], sem_o[b])

        def wait_out(b):
            pltpu.make_async_copy(
                cmps[b], out_hbm.at[pl.ds(base, CHUNK)], sem_o[b]).wait()

        start_gather(0, 0)

        @pl.loop(0, N_CHUNKS, step=2)
        def _(c):
            start_gather(c + 1, 1)
            wait_gather(0)

            @pl.when(c >= 2)
            def _():
                wait_out(0)
            compact(0)
            start_out(c, 0)

            @pl.when(c + 2 < N_CHUNKS)
            def _():
                start_gather(c + 2, 0)
            wait_gather(1)

            @pl.when(c >= 2)
            def _():
                wait_out(1)
            compact(1)
            start_out(c + 1, 1)

        wait_out(0)
        wait_out(1)

    return gather_kernel(table_p, flat_code)


PAD_ROWS = 2000


def _pad_table(table):
    def pad_kernel(t_ref, o_ref):
        o_ref[:, :TOKEN_DIM] = t_ref[...]
        o_ref[:, TOKEN_DIM:] = jnp.zeros((PAD_ROWS, PAD_DIM - TOKEN_DIM),
                                         jnp.float32)

    return pl.pallas_call(
        pad_kernel,
        grid=(VOCAB // PAD_ROWS,),
        in_specs=[pl.BlockSpec((PAD_ROWS, TOKEN_DIM), lambda i: (i, 0))],
        out_specs=pl.BlockSpec((PAD_ROWS, PAD_DIM), lambda i: (i, 0)),
        out_shape=jax.ShapeDtypeStruct((VOCAB, PAD_DIM), jnp.float32),
    )(table)


from jax.experimental import layout as jex_layout


def kernel(code, mask, table):
    flat_code = code.reshape(NUM_IDX)
    table_p = jnp.pad(table, ((0, 0), (0, PAD_DIM - TOKEN_DIM)))
    table_p = jex_layout.with_layout_constraint(
        table_p, jex_layout.Layout(major_to_minor=(0, 1), tiling=((8,),)))
    out3 = _sc_gather(table_p, flat_code)
    return out3, mask


# speedup vs baseline: 1.1924x; 1.1924x over previous
import jax
import jax.numpy as jnp
from jax import lax
from jax.experimental import pallas as pl
from jax.experimental.pallas import tpu as pltpu
from jax.experimental.pallas import tpu_sc as plsc

BATCH = 4096
SEQ = 200
TOKEN_DIM = 64
VOCAB = 100000
PAD_DIM = 128
NUM_IDX = BATCH * SEQ
NUM_WORKERS = 32
PER_WORKER = NUM_IDX // NUM_WORKERS
CHUNK = 160
N_CHUNKS = PER_WORKER // CHUNK
LANES = 16


def _sc_gather(table_p, flat_code):
    mesh = plsc.VectorSubcoreMesh(core_axis_name="c", subcore_axis_name="s")

    @pl.kernel(
        out_type=jax.ShapeDtypeStruct((NUM_IDX, TOKEN_DIM), table_p.dtype),
        mesh=mesh,
        scratch_types=[
            pltpu.VMEM((PER_WORKER,), jnp.int32),
            pltpu.VMEM((CHUNK, PAD_DIM), jnp.float32),
            pltpu.VMEM((CHUNK, PAD_DIM), jnp.float32),
            pltpu.VMEM((CHUNK, TOKEN_DIM), jnp.float32),
            pltpu.VMEM((CHUNK, TOKEN_DIM), jnp.float32),
            pltpu.SemaphoreType.DMA,
            pltpu.SemaphoreType.DMA,
            pltpu.SemaphoreType.DMA,
            pltpu.SemaphoreType.DMA,
        ],
    )
    def gather_kernel(table_hbm, idx_hbm, out_hbm, idx_all,
                      rows0, rows1, cmp0, cmp1,
                      sem_g0, sem_g1, sem_o0, sem_o1):
        wid = lax.axis_index("s") * 2 + lax.axis_index("c")
        base = wid * PER_WORKER
        rows = (rows0, rows1)
        cmps = (cmp0, cmp1)
        sem_g = (sem_g0, sem_g1)
        sem_o = (sem_o0, sem_o1)

        pltpu.sync_copy(idx_hbm.at[pl.ds(base, PER_WORKER)], idx_all)

        def start_gather(c, b):
            pltpu.async_copy(
                table_hbm.at[idx_all.at[pl.ds(c * CHUNK, CHUNK)]],
                rows[b], sem_g[b])

        def wait_gather(b):
            pltpu.make_async_copy(
                table_hbm.at[idx_all.at[pl.ds(0, CHUNK)]],
                rows[b], sem_g[b]).wait()

        def compact(b):
            @plsc.parallel_loop(0, CHUNK, unroll=4)
            def _(i):
                for k in range(TOKEN_DIM // LANES):
                    cmps[b][i, pl.ds(k * LANES, LANES)] = (
                        rows[b][i, pl.ds(k * LANES, LANES)])

        def start_out(c, b):
            pltpu.async_copy(cmps[b], out_hbm.at[pl.ds(base + c * CHUNK,
                                                       CHUNK)], sem_o[b])

        def wait_out(b):
            pltpu.make_async_copy(
                cmps[b], out_hbm.at[pl.ds(base, CHUNK)], sem_o[b]).wait()

        start_gather(0, 0)

        @pl.loop(0, N_CHUNKS, step=2)
        def _(c):
            start_gather(c + 1, 1)
            wait_gather(0)

            @pl.when(c >= 2)
            def _():
                wait_out(0)
            compact(0)
            start_out(c, 0)

            @pl.when(c + 2 < N_CHUNKS)
            def _():
                start_gather(c + 2, 0)
            wait_gather(1)

            @pl.when(c >= 2)
            def _():
                wait_out(1)
            compact(1)
            start_out(c + 1, 1)

        wait_out(0)
        wait_out(1)

    return gather_kernel(table_p, flat_code)


PAD_ROWS = 2000


def _pad_table(table):
    def pad_kernel(t_ref, o_ref):
        o_ref[:, :TOKEN_DIM] = t_ref[...]
        o_ref[:, TOKEN_DIM:] = jnp.zeros((PAD_ROWS, PAD_DIM - TOKEN_DIM),
                                         jnp.float32)

    return pl.pallas_call(
        pad_kernel,
        grid=(VOCAB // PAD_ROWS,),
        in_specs=[pl.BlockSpec((PAD_ROWS, TOKEN_DIM), lambda i: (i, 0))],
        out_specs=pl.BlockSpec((PAD_ROWS, PAD_DIM), lambda i: (i, 0)),
        out_shape=jax.ShapeDtypeStruct((VOCAB, PAD_DIM), jnp.float32),
    )(table)


from jax.experimental import layout as jex_layout


def kernel(code, mask, table):
    flat_code = code.reshape(NUM_IDX)
    table_p = jnp.pad(table, ((0, 0), (0, PAD_DIM - TOKEN_DIM)))
    table_p = jex_layout.with_layout_constraint(
        table_p, jex_layout.Layout(major_to_minor=(0, 1), tiling=((8,),)))
    out = _sc_gather(table_p, flat_code)
    return out.reshape(BATCH, SEQ, TOKEN_DIM), mask


# speedup vs baseline: 1.1986x; 1.0052x over previous
import jax
import jax.numpy as jnp
from jax import lax
from jax.experimental import pallas as pl
from jax.experimental.pallas import tpu as pltpu
from jax.experimental.pallas import tpu_sc as plsc

BATCH = 4096
SEQ = 200
TOKEN_DIM = 64
VOCAB = 100000
PAD_DIM = 128
NUM_IDX = BATCH * SEQ
NUM_WORKERS = 32
PER_WORKER = NUM_IDX // NUM_WORKERS
CHUNK = 200
N_CHUNKS = PER_WORKER // CHUNK
LANES = 16


def _sc_gather(table_p, flat_code):
    mesh = plsc.VectorSubcoreMesh(core_axis_name="c", subcore_axis_name="s")

    @pl.kernel(
        out_type=jax.ShapeDtypeStruct((NUM_IDX, TOKEN_DIM), table_p.dtype),
        mesh=mesh,
        scratch_types=[
            pltpu.VMEM((CHUNK,), jnp.int32),
            pltpu.VMEM((CHUNK,), jnp.int32),
            pltpu.VMEM((CHUNK, PAD_DIM), jnp.float32),
            pltpu.VMEM((CHUNK, PAD_DIM), jnp.float32),
            pltpu.VMEM((CHUNK, TOKEN_DIM), jnp.float32),
            pltpu.VMEM((CHUNK, TOKEN_DIM), jnp.float32),
            pltpu.SemaphoreType.DMA,
            pltpu.SemaphoreType.DMA,
            pltpu.SemaphoreType.DMA,
            pltpu.SemaphoreType.DMA,
        ],
    )
    def gather_kernel(table_hbm, idx_hbm, out_hbm, idx0, idx1,
                      rows0, rows1, cmp0, cmp1,
                      sem_g0, sem_g1, sem_o0, sem_o1):
        wid = lax.axis_index("s") * 2 + lax.axis_index("c")
        base = wid * PER_WORKER
        idxs = (idx0, idx1)
        rows = (rows0, rows1)
        cmps = (cmp0, cmp1)
        sem_g = (sem_g0, sem_g1)
        sem_o = (sem_o0, sem_o1)

        def start_gather(c, b):
            pltpu.sync_copy(idx_hbm.at[pl.ds(base + c * CHUNK, CHUNK)],
                            idxs[b])
            pltpu.async_copy(table_hbm.at[idxs[b]], rows[b], sem_g[b])

        def wait_gather(b):
            pltpu.make_async_copy(
                table_hbm.at[idxs[b]], rows[b], sem_g[b]).wait()

        def compact(b):
            @plsc.parallel_loop(0, CHUNK, unroll=4)
            def _(i):
                for k in range(TOKEN_DIM // LANES):
                    cmps[b][i, pl.ds(k * LANES, LANES)] = (
                        rows[b][i, pl.ds(k * LANES, LANES)])

        def start_out(c, b):
            pltpu.async_copy(cmps[b], out_hbm.at[pl.ds(base + c * CHUNK,
                                                       CHUNK)], sem_o[b])

        def wait_out(b):
            pltpu.make_async_copy(
                cmps[b], out_hbm.at[pl.ds(base, CHUNK)], sem_o[b]).wait()

        start_gather(0, 0)

        @pl.loop(0, N_CHUNKS, step=2)
        def _(c):
            start_gather(c + 1, 1)
            wait_gather(0)

            @pl.when(c >= 2)
            def _():
                wait_out(0)
            compact(0)
            start_out(c, 0)

            @pl.when(c + 2 < N_CHUNKS)
            def _():
                start_gather(c + 2, 0)
            wait_gather(1)

            @pl.when(c >= 2)
            def _():
                wait_out(1)
            compact(1)
            start_out(c + 1, 1)

        wait_out(0)
        wait_out(1)

    return gather_kernel(table_p, flat_code)


PAD_ROWS = 2000


def _pad_table(table):
    def pad_kernel(t_ref, o_ref):
        o_ref[:, :TOKEN_DIM] = t_ref[...]
        o_ref[:, TOKEN_DIM:] = jnp.zeros((PAD_ROWS, PAD_DIM - TOKEN_DIM),
                                         jnp.float32)

    return pl.pallas_call(
        pad_kernel,
        grid=(VOCAB // PAD_ROWS,),
        in_specs=[pl.BlockSpec((PAD_ROWS, TOKEN_DIM), lambda i: (i, 0))],
        out_specs=pl.BlockSpec((PAD_ROWS, PAD_DIM), lambda i: (i, 0)),
        out_shape=jax.ShapeDtypeStruct((VOCAB, PAD_DIM), jnp.float32),
    )(table)


from jax.experimental import layout as jex_layout


def kernel(code, mask, table):
    flat_code = code.reshape(NUM_IDX)
    table_p = jnp.pad(table, ((0, 0), (0, PAD_DIM - TOKEN_DIM)))
    table_p = jex_layout.with_layout_constraint(
        table_p, jex_layout.Layout(major_to_minor=(0, 1), tiling=((8,),)))
    out = _sc_gather(table_p, flat_code)
    return out.reshape(BATCH, SEQ, TOKEN_DIM), mask


# speedup vs baseline: 1.2031x; 1.0038x over previous
import jax
import jax.numpy as jnp
from jax import lax
from jax.experimental import layout as jex_layout
from jax.experimental import pallas as pl
from jax.experimental.pallas import tpu as pltpu
from jax.experimental.pallas import tpu_sc as plsc

BATCH = 4096
SEQ = 200
TOKEN_DIM = 64
VOCAB = 100000
PAD_DIM = 128
NUM_IDX = BATCH * SEQ
NUM_WORKERS = 32
PER_WORKER = NUM_IDX // NUM_WORKERS
CHUNK = 200
N_CHUNKS = PER_WORKER // CHUNK
LANES = 16


def _sc_gather(table_p, flat_code):
    mesh = plsc.VectorSubcoreMesh(core_axis_name="c", subcore_axis_name="s")

    @pl.kernel(
        out_type=jax.ShapeDtypeStruct((NUM_IDX, TOKEN_DIM), table_p.dtype),
        mesh=mesh,
        scratch_types=[
            pltpu.VMEM((CHUNK,), jnp.int32),
            pltpu.VMEM((CHUNK,), jnp.int32),
            pltpu.VMEM((CHUNK, PAD_DIM), jnp.float32),
            pltpu.VMEM((CHUNK, PAD_DIM), jnp.float32),
            pltpu.VMEM((CHUNK, TOKEN_DIM), jnp.float32),
            pltpu.VMEM((CHUNK, TOKEN_DIM), jnp.float32),
            pltpu.SemaphoreType.DMA,
            pltpu.SemaphoreType.DMA,
            pltpu.SemaphoreType.DMA,
            pltpu.SemaphoreType.DMA,
        ],
    )
    def gather_kernel(table_hbm, idx_hbm, out_hbm, idx0, idx1,
                      rows0, rows1, cmp0, cmp1,
                      sem_g0, sem_g1, sem_o0, sem_o1):
        wid = lax.axis_index("s") * 2 + lax.axis_index("c")
        base = wid * PER_WORKER
        idxs = (idx0, idx1)
        rows = (rows0, rows1)
        cmps = (cmp0, cmp1)
        sem_g = (sem_g0, sem_g1)
        sem_o = (sem_o0, sem_o1)

        def start_gather(c, b):
            pltpu.sync_copy(idx_hbm.at[pl.ds(base + c * CHUNK, CHUNK)],
                            idxs[b])
            pltpu.async_copy(table_hbm.at[idxs[b]], rows[b], sem_g[b])

        def wait_gather(b):
            pltpu.make_async_copy(
                table_hbm.at[idxs[b]], rows[b], sem_g[b]).wait()

        def compact(b):
            @plsc.parallel_loop(0, CHUNK, unroll=4)
            def _(i):
                for k in range(TOKEN_DIM // LANES):
                    cmps[b][i, pl.ds(k * LANES, LANES)] = (
                        rows[b][i, pl.ds(k * LANES, LANES)])

        def start_out(c, b):
            pltpu.async_copy(cmps[b], out_hbm.at[pl.ds(base + c * CHUNK,
                                                       CHUNK)], sem_o[b])

        def wait_out(b):
            pltpu.make_async_copy(
                cmps[b], out_hbm.at[pl.ds(base, CHUNK)], sem_o[b]).wait()

        start_gather(0, 0)

        @pl.loop(0, N_CHUNKS, step=2)
        def _(c):
            start_gather(c + 1, 1)
            wait_gather(0)

            @pl.when(c >= 2)
            def _():
                wait_out(0)
            compact(0)
            start_out(c, 0)

            @pl.when(c + 2 < N_CHUNKS)
            def _():
                start_gather(c + 2, 0)
            wait_gather(1)

            @pl.when(c >= 2)
            def _():
                wait_out(1)
            compact(1)
            start_out(c + 1, 1)

        wait_out(0)
        wait_out(1)

    return gather_kernel(table_p, flat_code)


def kernel(code, mask, table):
    flat_code = code.reshape(NUM_IDX)
    table_p = jnp.pad(table, ((0, 0), (0, PAD_DIM - TOKEN_DIM)))
    table_p = jex_layout.with_layout_constraint(
        table_p, jex_layout.Layout(major_to_minor=(0, 1), tiling=((8,),)))
    out = _sc_gather(table_p, flat_code)
    return out.reshape(BATCH, SEQ, TOKEN_DIM), mask
